# baseline (speedup 1.0000x reference)
import jax
import jax.numpy as jnp
from jax.experimental import pallas as pl
from jax.experimental.pallas import tpu as pltpu

N = 20000
R, C = 160, 128
P = R * C
POOL_ROWS = 8
IMG_H, IMG_W = 512.0, 512.0
BOX_FILTER_THRESHOLD = 0.05
NMS_THRESHOLD = 0.5
POST_NMS_TOP_K = 100
NEG_INF = -1e9


def _nms_kernel(cls_ref, del_ref, anc_ref,
                box_out, sc_out, lb_out,
                x1_ref, y1_ref, x2_ref, y2_ref, area_ref, lab_ref, sw_ref):
    num_classes = cls_ref.shape[1]

    row_iota = jax.lax.broadcasted_iota(jnp.int32, (R, C), 0)
    col_iota = jax.lax.broadcasted_iota(jnp.int32, (R, C), 1)
    lin = row_iota * C + col_iota
    lane_iota = jax.lax.broadcasted_iota(jnp.int32, (1, C), 1)
    slin = (jax.lax.broadcasted_iota(jnp.int32, (8, 128), 0) * 128
            + jax.lax.broadcasted_iota(jnp.int32, (8, 128), 1))
    cls_iota = jax.lax.broadcasted_iota(jnp.int32, (num_classes, C), 0)

    def fmt_body(k, carry):
        blk = cls_ref[pl.ds(k * 128, 128), :]
        blk_t = blk.T
        m = jnp.max(blk_t, axis=0, keepdims=True)
        lab = jnp.min(jnp.where(blk_t == m, cls_iota, num_classes),
                      axis=0, keepdims=True)
        sw_ref[pl.ds(k, 1), :] = jnp.where(m >= BOX_FILTER_THRESHOLD,
                                           m, NEG_INF)
        lab_ref[pl.ds(k, 1), :] = lab
        return carry

    jax.lax.fori_loop(0, R, fmt_body, 0)

    ax, ay, aw, ah = anc_ref[0], anc_ref[1], anc_ref[2], anc_ref[3]
    dx, dy, dw, dh = del_ref[0], del_ref[1], del_ref[2], del_ref[3]
    cx = ax + dx * aw
    cy = ay + dy * ah
    w = aw * jnp.exp(dw)
    h = ah * jnp.exp(dh)
    x1 = jnp.clip(cx - 0.5 * w, 0.0, IMG_W)
    y1 = jnp.clip(cy - 0.5 * h, 0.0, IMG_H)
    x2 = jnp.clip(cx + 0.5 * w, 0.0, IMG_W)
    y2 = jnp.clip(cy + 0.5 * h, 0.0, IMG_H)
    area = jnp.maximum(x2 - x1, 0.0) * jnp.maximum(y2 - y1, 0.0)
    x1_ref[...] = x1
    y1_ref[...] = y1
    x2_ref[...] = x2
    y2_ref[...] = y2
    area_ref[...] = area

    sc_out[...] = jnp.zeros((8, 128), jnp.float32)
    lb_out[...] = jnp.full((8, 128), -1, jnp.int32)
    for i in range(4):
        box_out[i] = jnp.zeros((8, 128), jnp.float32)

    swv = sw_ref[...]
    labv = lab_ref[...]
    work = swv
    prows = {k: [] for k in ('sw', 'idx', 'x1', 'y1', 'x2', 'y2', 'a', 'l')}
    for _ in range(POOL_ROWS):
        m = jnp.max(work, axis=0, keepdims=True)
        sel_row = jnp.min(jnp.where(work == m, row_iota, R),
                          axis=0, keepdims=True)
        mask = row_iota == sel_row
        prows['sw'].append(m)
        prows['idx'].append(sel_row * C + lane_iota)
        prows['x1'].append(jnp.sum(jnp.where(mask, x1, 0.0), axis=0,
                                   keepdims=True))
        prows['y1'].append(jnp.sum(jnp.where(mask, y1, 0.0), axis=0,
                                   keepdims=True))
        prows['x2'].append(jnp.sum(jnp.where(mask, x2, 0.0), axis=0,
                                   keepdims=True))
        prows['y2'].append(jnp.sum(jnp.where(mask, y2, 0.0), axis=0,
                                   keepdims=True))
        prows['a'].append(jnp.sum(jnp.where(mask, area, 0.0), axis=0,
                                  keepdims=True))
        prows['l'].append(jnp.sum(jnp.where(mask, labv, 0), axis=0,
                                  keepdims=True))
        work = jnp.where(mask, -jnp.inf, work)
    psw0 = jnp.concatenate(prows['sw'], axis=0)
    pidx = jnp.concatenate(prows['idx'], axis=0)
    px1 = jnp.concatenate(prows['x1'], axis=0)
    py1 = jnp.concatenate(prows['y1'], axis=0)
    px2 = jnp.concatenate(prows['x2'], axis=0)
    py2 = jnp.concatenate(prows['y2'], axis=0)
    parea = jnp.concatenate(prows['a'], axis=0)
    plab = jnp.concatenate(prows['l'], axis=0)

    def red2(v, op):
        return op(op(v, axis=0, keepdims=True), axis=1, keepdims=True)

    tau = red2(work, jnp.max)
    tau_live = tau > (NEG_INF / 2.0)

    def pool_body(t, carry):
        psw, bad = carry
        s = red2(psw, jnp.max)
        pick = red2(jnp.where(psw == s, pidx, jnp.int32(P)), jnp.min)
        hot = pidx == pick
        bx1 = red2(jnp.where(hot, px1, 0.0), jnp.sum)
        by1 = red2(jnp.where(hot, py1, 0.0), jnp.sum)
        bx2 = red2(jnp.where(hot, px2, 0.0), jnp.sum)
        by2 = red2(jnp.where(hot, py2, 0.0), jnp.sum)
        blab = red2(jnp.where(hot, plab, 0), jnp.sum)
        area_a = jnp.maximum(bx2 - bx1, 0.0) * jnp.maximum(by2 - by1, 0.0)
        valid = s > (NEG_INF / 2.0)

        inter = (jnp.maximum(jnp.minimum(bx2, px2) - jnp.maximum(bx1, px1),
                             0.0)
                 * jnp.maximum(jnp.minimum(by2, py2) - jnp.maximum(by1, py1),
                               0.0))
        iou = inter / (area_a + parea - inter + 1e-9)
        psw = jnp.where(((iou > NMS_THRESHOLD) & valid) | hot, NEG_INF, psw)

        wr = (slin == t) & valid
        sc_out[...] = jnp.where(wr, s, sc_out[...])
        lb_out[...] = jnp.where(wr, blab, lb_out[...])
        bvals = (bx1, by1, bx2, by2)
        for i in range(4):
            box_out[i] = jnp.where(wr, bvals[i], box_out[i])

        bad = jnp.where((s <= tau) & tau_live, 1.0, bad)
        return psw, bad

    _, badf = jax.lax.fori_loop(0, POST_NMS_TOP_K, pool_body,
                                (psw0, jnp.zeros((1, 1), jnp.float32)))

    @pl.when(badf[0, 0] > 0.5)
    def _fallback():
        sc_out[...] = jnp.zeros((8, 128), jnp.float32)
        lb_out[...] = jnp.full((8, 128), -1, jnp.int32)
        for i in range(4):
            box_out[i] = jnp.zeros((8, 128), jnp.float32)

        def full_body(t, sw):
            s = jnp.max(sw)
            idx = jnp.min(jnp.where(sw == s, lin, jnp.int32(P)))
            row = idx // C
            lane_hot = lane_iota == idx - row * C

            def ext(ref, zero):
                return jnp.sum(jnp.where(lane_hot, ref[pl.ds(row, 1), :],
                                         zero))

            bx1 = ext(x1_ref, 0.0)
            by1 = ext(y1_ref, 0.0)
            bx2 = ext(x2_ref, 0.0)
            by2 = ext(y2_ref, 0.0)
            blab = ext(lab_ref, 0)
            area_a = jnp.maximum(bx2 - bx1, 0.0) * jnp.maximum(by2 - by1, 0.0)
            valid = s > (NEG_INF / 2.0)

            inter = (jnp.maximum(jnp.minimum(bx2, x2_ref[...])
                                 - jnp.maximum(bx1, x1_ref[...]), 0.0)
                     * jnp.maximum(jnp.minimum(by2, y2_ref[...])
                                   - jnp.maximum(by1, y1_ref[...]), 0.0))
            iou = inter / (area_a + area_ref[...] - inter + 1e-9)
            sw = jnp.where(((iou > NMS_THRESHOLD) & valid) | (lin == idx),
                           NEG_INF, sw)

            wr = (slin == t) & valid
            sc_out[...] = jnp.where(wr, s, sc_out[...])
            lb_out[...] = jnp.where(wr, blab, lb_out[...])
            bvals = (bx1, by1, bx2, by2)
            for i in range(4):
                box_out[i] = jnp.where(wr, bvals[i], box_out[i])
            return sw

        jax.lax.fori_loop(0, POST_NMS_TOP_K, full_body, sw_ref[...])


def kernel(cls_scores, box_deltas, anchors):
    n, num_classes = cls_scores.shape
    pad = P - n
    cls_p = jnp.pad(cls_scores, ((0, pad), (0, 0)), constant_values=-1.0)
    del_t = jnp.pad(box_deltas, ((0, pad), (0, 0))).T.reshape(4, R, C)
    anc_t = jnp.pad(anchors, ((0, pad), (0, 0))).T.reshape(4, R, C)

    f32, i32 = jnp.float32, jnp.int32
    bx, sc, lb = pl.pallas_call(
        _nms_kernel,
        out_shape=(
            jax.ShapeDtypeStruct((4, 8, 128), f32),
            jax.ShapeDtypeStruct((8, 128), f32),
            jax.ShapeDtypeStruct((8, 128), i32),
        ),
        scratch_shapes=[
            pltpu.VMEM((R, C), f32),
            pltpu.VMEM((R, C), f32),
            pltpu.VMEM((R, C), f32),
            pltpu.VMEM((R, C), f32),
            pltpu.VMEM((R, C), f32),
            pltpu.VMEM((R, C), i32),
            pltpu.VMEM((R, C), f32),
        ],
    )(cls_p, del_t, anc_t)

    boxes = bx.reshape(4, 8 * 128)[:, :POST_NMS_TOP_K].T
    scores = sc.reshape(8 * 128)[:POST_NMS_TOP_K]
    labels = lb.reshape(8 * 128)[:POST_NMS_TOP_K]
    return boxes, scores, labels


# speedup vs baseline: 1.5578x; 1.5578x over previous
import jax
import jax.numpy as jnp
from jax.experimental import pallas as pl
from jax.experimental.pallas import tpu as pltpu

N = 20000
R, C = 160, 128
P = R * C
POOL_ROWS = 8
IMG_H, IMG_W = 512.0, 512.0
BOX_FILTER_THRESHOLD = 0.05
NMS_THRESHOLD = 0.5
POST_NMS_TOP_K = 100
NEG_INF = -1e9


def _nms_kernel(cls_ref, del_ref, anc_ref,
                box_out, sc_out, lb_out,
                x1_ref, y1_ref, x2_ref, y2_ref, area_ref, lab_ref, sw_ref):
    num_classes = cls_ref.shape[0]

    row_iota = jax.lax.broadcasted_iota(jnp.int32, (R, C), 0)
    col_iota = jax.lax.broadcasted_iota(jnp.int32, (R, C), 1)
    lin = row_iota * C + col_iota
    lane_iota = jax.lax.broadcasted_iota(jnp.int32, (1, C), 1)
    slin = (jax.lax.broadcasted_iota(jnp.int32, (8, 128), 0) * 128
            + jax.lax.broadcasted_iota(jnp.int32, (8, 128), 1))

    def cls_body(c, carry):
        best, lab = carry
        v = cls_ref[c]
        better = v > best
        return jnp.where(better, v, best), jnp.where(better, c, lab)

    best, labv = jax.lax.fori_loop(
        1, num_classes, cls_body, (cls_ref[0], jnp.zeros((R, C), jnp.int32)))
    lab_ref[...] = labv

    ax, ay, aw, ah = anc_ref[0], anc_ref[1], anc_ref[2], anc_ref[3]
    dx, dy, dw, dh = del_ref[0], del_ref[1], del_ref[2], del_ref[3]
    cx = ax + dx * aw
    cy = ay + dy * ah
    w = aw * jnp.exp(dw)
    h = ah * jnp.exp(dh)
    x1 = jnp.clip(cx - 0.5 * w, 0.0, IMG_W)
    y1 = jnp.clip(cy - 0.5 * h, 0.0, IMG_H)
    x2 = jnp.clip(cx + 0.5 * w, 0.0, IMG_W)
    y2 = jnp.clip(cy + 0.5 * h, 0.0, IMG_H)
    area = jnp.maximum(x2 - x1, 0.0) * jnp.maximum(y2 - y1, 0.0)
    x1_ref[...] = x1
    y1_ref[...] = y1
    x2_ref[...] = x2
    y2_ref[...] = y2
    area_ref[...] = area

    swv = jnp.where(best >= BOX_FILTER_THRESHOLD, best, NEG_INF)
    sw_ref[...] = swv

    work = swv
    prows = {k: [] for k in ('sw', 'idx', 'x1', 'y1', 'x2', 'y2', 'a', 'l')}
    for _ in range(POOL_ROWS):
        m = jnp.max(work, axis=0, keepdims=True)
        sel_row = jnp.min(jnp.where(work == m, row_iota, R),
                          axis=0, keepdims=True)
        mask = row_iota == sel_row
        prows['sw'].append(m)
        prows['idx'].append(sel_row * C + lane_iota)
        prows['x1'].append(jnp.sum(jnp.where(mask, x1, 0.0), axis=0,
                                   keepdims=True))
        prows['y1'].append(jnp.sum(jnp.where(mask, y1, 0.0), axis=0,
                                   keepdims=True))
        prows['x2'].append(jnp.sum(jnp.where(mask, x2, 0.0), axis=0,
                                   keepdims=True))
        prows['y2'].append(jnp.sum(jnp.where(mask, y2, 0.0), axis=0,
                                   keepdims=True))
        prows['a'].append(jnp.sum(jnp.where(mask, area, 0.0), axis=0,
                                  keepdims=True))
        prows['l'].append(jnp.sum(jnp.where(mask, labv, 0), axis=0,
                                  keepdims=True))
        work = jnp.where(mask, -jnp.inf, work)
    psw0 = jnp.concatenate(prows['sw'], axis=0)
    pidx = jnp.concatenate(prows['idx'], axis=0)
    px1 = jnp.concatenate(prows['x1'], axis=0)
    py1 = jnp.concatenate(prows['y1'], axis=0)
    px2 = jnp.concatenate(prows['x2'], axis=0)
    py2 = jnp.concatenate(prows['y2'], axis=0)
    parea = jnp.concatenate(prows['a'], axis=0)
    plab = jnp.concatenate(prows['l'], axis=0)

    def red2(v, op):
        return op(op(v, axis=0, keepdims=True), axis=1, keepdims=True)

    tau = red2(work, jnp.max)
    tau_live = tau > (NEG_INF / 2.0)

    zf = jnp.zeros((8, 128), jnp.float32)
    init = (psw0, jnp.zeros((1, 1), jnp.float32),
            zf, jnp.full((8, 128), -1, jnp.int32), zf, zf, zf, zf)

    def pool_body(t, carry):
        psw, bad, osc, olb, ob1, ob2, ob3, ob4 = carry
        s = red2(psw, jnp.max)
        pick = red2(jnp.where(psw == s, pidx, jnp.int32(P)), jnp.min)
        hot = pidx == pick
        bx1 = red2(jnp.where(hot, px1, 0.0), jnp.sum)
        by1 = red2(jnp.where(hot, py1, 0.0), jnp.sum)
        bx2 = red2(jnp.where(hot, px2, 0.0), jnp.sum)
        by2 = red2(jnp.where(hot, py2, 0.0), jnp.sum)
        blab = red2(jnp.where(hot, plab, 0), jnp.sum)
        area_a = red2(jnp.where(hot, parea, 0.0), jnp.sum)
        valid = s > (NEG_INF / 2.0)

        inter = (jnp.maximum(jnp.minimum(bx2, px2) - jnp.maximum(bx1, px1),
                             0.0)
                 * jnp.maximum(jnp.minimum(by2, py2) - jnp.maximum(by1, py1),
                               0.0))
        iou = inter / (area_a + parea - inter + 1e-9)
        psw = jnp.where(((iou > NMS_THRESHOLD) & valid) | hot, NEG_INF, psw)

        wr = (slin == t) & valid
        osc = jnp.where(wr, s, osc)
        olb = jnp.where(wr, blab, olb)
        ob1 = jnp.where(wr, bx1, ob1)
        ob2 = jnp.where(wr, by1, ob2)
        ob3 = jnp.where(wr, bx2, ob3)
        ob4 = jnp.where(wr, by2, ob4)
        bad = jnp.where((s <= tau) & tau_live, 1.0, bad)
        return psw, bad, osc, olb, ob1, ob2, ob3, ob4

    (_, badf, osc, olb, ob1, ob2, ob3, ob4) = jax.lax.fori_loop(
        0, POST_NMS_TOP_K, pool_body, init)

    sc_out[...] = osc
    lb_out[...] = olb
    for i, ob in enumerate((ob1, ob2, ob3, ob4)):
        box_out[i] = ob

    @pl.when(badf[0, 0] > 0.5)
    def _fallback():
        sc_out[...] = jnp.zeros((8, 128), jnp.float32)
        lb_out[...] = jnp.full((8, 128), -1, jnp.int32)
        for i in range(4):
            box_out[i] = jnp.zeros((8, 128), jnp.float32)

        def full_body(t, sw):
            s = jnp.max(sw)
            idx = jnp.min(jnp.where(sw == s, lin, jnp.int32(P)))
            row = idx // C
            lane_hot = lane_iota == idx - row * C

            def ext(ref, zero):
                return jnp.sum(jnp.where(lane_hot, ref[pl.ds(row, 1), :],
                                         zero))

            bx1 = ext(x1_ref, 0.0)
            by1 = ext(y1_ref, 0.0)
            bx2 = ext(x2_ref, 0.0)
            by2 = ext(y2_ref, 0.0)
            blab = ext(lab_ref, 0)
            area_a = jnp.maximum(bx2 - bx1, 0.0) * jnp.maximum(by2 - by1, 0.0)
            valid = s > (NEG_INF / 2.0)

            inter = (jnp.maximum(jnp.minimum(bx2, x2_ref[...])
                                 - jnp.maximum(bx1, x1_ref[...]), 0.0)
                     * jnp.maximum(jnp.minimum(by2, y2_ref[...])
                                   - jnp.maximum(by1, y1_ref[...]), 0.0))
            iou = inter / (area_a + area_ref[...] - inter + 1e-9)
            sw = jnp.where(((iou > NMS_THRESHOLD) & valid) | (lin == idx),
                           NEG_INF, sw)

            wr = (slin == t) & valid
            sc_out[...] = jnp.where(wr, s, sc_out[...])
            lb_out[...] = jnp.where(wr, blab, lb_out[...])
            bvals = (bx1, by1, bx2, by2)
            for i in range(4):
                box_out[i] = jnp.where(wr, bvals[i], box_out[i])
            return sw

        jax.lax.fori_loop(0, POST_NMS_TOP_K, full_body, sw_ref[...])


def kernel(cls_scores, box_deltas, anchors):
    n, num_classes = cls_scores.shape
    pad = P - n
    cls_t = jnp.pad(cls_scores, ((0, pad), (0, 0)),
                    constant_values=-1.0).T.reshape(num_classes, R, C)
    del_t = jnp.pad(box_deltas, ((0, pad), (0, 0))).T.reshape(4, R, C)
    anc_t = jnp.pad(anchors, ((0, pad), (0, 0))).T.reshape(4, R, C)

    f32, i32 = jnp.float32, jnp.int32
    bx, sc, lb = pl.pallas_call(
        _nms_kernel,
        out_shape=(
            jax.ShapeDtypeStruct((4, 8, 128), f32),
            jax.ShapeDtypeStruct((8, 128), f32),
            jax.ShapeDtypeStruct((8, 128), i32),
        ),
        scratch_shapes=[
            pltpu.VMEM((R, C), f32),
            pltpu.VMEM((R, C), f32),
            pltpu.VMEM((R, C), f32),
            pltpu.VMEM((R, C), f32),
            pltpu.VMEM((R, C), f32),
            pltpu.VMEM((R, C), i32),
            pltpu.VMEM((R, C), f32),
        ],
    )(cls_t, del_t, anc_t)

    boxes = bx.reshape(4, 8 * 128)[:, :POST_NMS_TOP_K].T
    scores = sc.reshape(8 * 128)[:POST_NMS_TOP_K]
    labels = lb.reshape(8 * 128)[:POST_NMS_TOP_K]
    return boxes, scores, labels


# speedup vs baseline: 1.5613x; 1.0022x over previous
import jax
import jax.numpy as jnp
from jax.experimental import pallas as pl
from jax.experimental.pallas import tpu as pltpu

N = 20000
R, C = 160, 128
P = R * C
POOL_ROWS = 8
IMG_H, IMG_W = 512.0, 512.0
BOX_FILTER_THRESHOLD = 0.05
NMS_THRESHOLD = 0.5
POST_NMS_TOP_K = 100
NEG_INF = -1e9


def _nms_kernel(cls_ref, del_ref, anc_ref,
                box_out, sc_out, lb_out,
                x1_ref, y1_ref, x2_ref, y2_ref, area_ref, lab_ref, sw_ref):
    num_classes = cls_ref.shape[0]

    row_iota = jax.lax.broadcasted_iota(jnp.int32, (R, C), 0)
    col_iota = jax.lax.broadcasted_iota(jnp.int32, (R, C), 1)
    lin = row_iota * C + col_iota
    lane_iota = jax.lax.broadcasted_iota(jnp.int32, (1, C), 1)
    slin = (jax.lax.broadcasted_iota(jnp.int32, (8, 128), 0) * 128
            + jax.lax.broadcasted_iota(jnp.int32, (8, 128), 1))

    def cls_body(c, carry):
        best, lab = carry
        v = cls_ref[c]
        better = v > best
        return jnp.where(better, v, best), jnp.where(better, c, lab)

    best, labv = jax.lax.fori_loop(
        1, num_classes, cls_body, (cls_ref[0], jnp.zeros((R, C), jnp.int32)))
    lab_ref[...] = labv

    ax, ay, aw, ah = anc_ref[0], anc_ref[1], anc_ref[2], anc_ref[3]
    dx, dy, dw, dh = del_ref[0], del_ref[1], del_ref[2], del_ref[3]
    cx = ax + dx * aw
    cy = ay + dy * ah
    w = aw * jnp.exp(dw)
    h = ah * jnp.exp(dh)
    x1 = jnp.clip(cx - 0.5 * w, 0.0, IMG_W)
    y1 = jnp.clip(cy - 0.5 * h, 0.0, IMG_H)
    x2 = jnp.clip(cx + 0.5 * w, 0.0, IMG_W)
    y2 = jnp.clip(cy + 0.5 * h, 0.0, IMG_H)
    area = jnp.maximum(x2 - x1, 0.0) * jnp.maximum(y2 - y1, 0.0)
    x1_ref[...] = x1
    y1_ref[...] = y1
    x2_ref[...] = x2
    y2_ref[...] = y2
    area_ref[...] = area

    swv = jnp.where(best >= BOX_FILTER_THRESHOLD, best, NEG_INF)
    sw_ref[...] = swv

    work = swv
    prows = {k: [] for k in ('sw', 'idx', 'x1', 'y1', 'x2', 'y2', 'a', 'l')}
    for _ in range(POOL_ROWS):
        m = jnp.max(work, axis=0, keepdims=True)
        sel_row = jnp.min(jnp.where(work == m, row_iota, R),
                          axis=0, keepdims=True)
        mask = row_iota == sel_row
        prows['sw'].append(m)
        prows['idx'].append(sel_row * C + lane_iota)
        prows['x1'].append(jnp.sum(jnp.where(mask, x1, 0.0), axis=0,
                                   keepdims=True))
        prows['y1'].append(jnp.sum(jnp.where(mask, y1, 0.0), axis=0,
                                   keepdims=True))
        prows['x2'].append(jnp.sum(jnp.where(mask, x2, 0.0), axis=0,
                                   keepdims=True))
        prows['y2'].append(jnp.sum(jnp.where(mask, y2, 0.0), axis=0,
                                   keepdims=True))
        prows['a'].append(jnp.sum(jnp.where(mask, area, 0.0), axis=0,
                                  keepdims=True))
        prows['l'].append(jnp.sum(jnp.where(mask, labv, 0), axis=0,
                                  keepdims=True))
        work = jnp.where(mask, -jnp.inf, work)
    psw0 = jnp.concatenate(prows['sw'], axis=0)
    pidx = jnp.concatenate(prows['idx'], axis=0)
    px1 = jnp.concatenate(prows['x1'], axis=0)
    py1 = jnp.concatenate(prows['y1'], axis=0)
    px2 = jnp.concatenate(prows['x2'], axis=0)
    py2 = jnp.concatenate(prows['y2'], axis=0)
    parea = jnp.concatenate(prows['a'], axis=0)
    plab = jnp.concatenate(prows['l'], axis=0)

    def red2(v, op):
        return op(op(v, axis=0, keepdims=True), axis=1, keepdims=True)

    tau = red2(work, jnp.max)
    tau_live = tau > (NEG_INF / 2.0)

    zf = jnp.zeros((8, 128), jnp.float32)
    init = (psw0, jnp.zeros((1, 1), jnp.float32),
            zf, jnp.full((8, 128), -1, jnp.int32), zf, zf, zf, zf)

    def pool_body(t, carry):
        psw, bad, osc, olb, ob1, ob2, ob3, ob4 = carry
        s = red2(psw, jnp.max)
        pick = red2(jnp.where(psw == s, pidx, jnp.int32(P)), jnp.min)
        hot = pidx == pick
        bx1 = red2(jnp.where(hot, px1, 0.0), jnp.sum)
        by1 = red2(jnp.where(hot, py1, 0.0), jnp.sum)
        bx2 = red2(jnp.where(hot, px2, 0.0), jnp.sum)
        by2 = red2(jnp.where(hot, py2, 0.0), jnp.sum)
        blab = red2(jnp.where(hot, plab, 0), jnp.sum)
        area_a = red2(jnp.where(hot, parea, 0.0), jnp.sum)
        valid = s > (NEG_INF / 2.0)

        inter = (jnp.maximum(jnp.minimum(bx2, px2) - jnp.maximum(bx1, px1),
                             0.0)
                 * jnp.maximum(jnp.minimum(by2, py2) - jnp.maximum(by1, py1),
                               0.0))
        iou = inter / (area_a + parea - inter + 1e-9)
        psw = jnp.where(((iou > NMS_THRESHOLD) & valid) | hot, NEG_INF, psw)

        wr = (slin == t) & valid
        osc = jnp.where(wr, s, osc)
        olb = jnp.where(wr, blab, olb)
        ob1 = jnp.where(wr, bx1, ob1)
        ob2 = jnp.where(wr, by1, ob2)
        ob3 = jnp.where(wr, bx2, ob3)
        ob4 = jnp.where(wr, by2, ob4)
        bad = jnp.where((s <= tau) & tau_live, 1.0, bad)
        return psw, bad, osc, olb, ob1, ob2, ob3, ob4

    (_, badf, osc, olb, ob1, ob2, ob3, ob4) = jax.lax.fori_loop(
        0, POST_NMS_TOP_K, pool_body, init)

    sc_out[...] = osc
    lb_out[...] = olb
    for i, ob in enumerate((ob1, ob2, ob3, ob4)):
        box_out[i] = ob

    @pl.when(badf[0, 0] > 0.5)
    def _fallback():
        sc_out[...] = jnp.zeros((8, 128), jnp.float32)
        lb_out[...] = jnp.full((8, 128), -1, jnp.int32)
        for i in range(4):
            box_out[i] = jnp.zeros((8, 128), jnp.float32)

        def full_body(t, sw):
            s = jnp.max(sw)
            idx = jnp.min(jnp.where(sw == s, lin, jnp.int32(P)))
            row = idx // C
            lane_hot = lane_iota == idx - row * C

            def ext(ref, zero):
                return jnp.sum(jnp.where(lane_hot, ref[pl.ds(row, 1), :],
                                         zero))

            bx1 = ext(x1_ref, 0.0)
            by1 = ext(y1_ref, 0.0)
            bx2 = ext(x2_ref, 0.0)
            by2 = ext(y2_ref, 0.0)
            blab = ext(lab_ref, 0)
            area_a = jnp.maximum(bx2 - bx1, 0.0) * jnp.maximum(by2 - by1, 0.0)
            valid = s > (NEG_INF / 2.0)

            inter = (jnp.maximum(jnp.minimum(bx2, x2_ref[...])
                                 - jnp.maximum(bx1, x1_ref[...]), 0.0)
                     * jnp.maximum(jnp.minimum(by2, y2_ref[...])
                                   - jnp.maximum(by1, y1_ref[...]), 0.0))
            iou = inter / (area_a + area_ref[...] - inter + 1e-9)
            sw = jnp.where(((iou > NMS_THRESHOLD) & valid) | (lin == idx),
                           NEG_INF, sw)

            wr = (slin == t) & valid
            sc_out[...] = jnp.where(wr, s, sc_out[...])
            lb_out[...] = jnp.where(wr, blab, lb_out[...])
            bvals = (bx1, by1, bx2, by2)
            for i in range(4):
                box_out[i] = jnp.where(wr, bvals[i], box_out[i])
            return sw

        jax.lax.fori_loop(0, POST_NMS_TOP_K, full_body, sw_ref[...])


def kernel(cls_scores, box_deltas, anchors):
    n, num_classes = cls_scores.shape
    pad = P - n
    cls_t = jnp.pad(cls_scores.T, ((0, 0), (0, pad)),
                    constant_values=-1.0).reshape(num_classes, R, C)
    del_t = jnp.pad(box_deltas.T, ((0, 0), (0, pad))).reshape(4, R, C)
    anc_t = jnp.pad(anchors.T, ((0, 0), (0, pad))).reshape(4, R, C)

    f32, i32 = jnp.float32, jnp.int32
    bx, sc, lb = pl.pallas_call(
        _nms_kernel,
        out_shape=(
            jax.ShapeDtypeStruct((4, 8, 128), f32),
            jax.ShapeDtypeStruct((8, 128), f32),
            jax.ShapeDtypeStruct((8, 128), i32),
        ),
        scratch_shapes=[
            pltpu.VMEM((R, C), f32),
            pltpu.VMEM((R, C), f32),
            pltpu.VMEM((R, C), f32),
            pltpu.VMEM((R, C), f32),
            pltpu.VMEM((R, C), f32),
            pltpu.VMEM((R, C), i32),
            pltpu.VMEM((R, C), f32),
        ],
    )(cls_t, del_t, anc_t)

    boxes = bx.reshape(4, 8 * 128)[:, :POST_NMS_TOP_K].T
    scores = sc.reshape(8 * 128)[:POST_NMS_TOP_K]
    labels = lb.reshape(8 * 128)[:POST_NMS_TOP_K]
    return boxes, scores, labels
